# initial kernel scaffold (unmeasured)
import jax
import jax.numpy as jnp
from jax import lax
from jax.experimental import pallas as pl
from jax.experimental.pallas import tpu as pltpu

N_DEV = 4

_sem_signal = getattr(pl, "semaphore_signal", None) or getattr(pltpu, "semaphore_signal")
_sem_wait = getattr(pl, "semaphore_wait", None) or getattr(pltpu, "semaphore_wait")
_DevIdType = getattr(pl, "DeviceIdType", None) or getattr(pltpu, "DeviceIdType")
_ANY = getattr(pltpu, "ANY", None)
if _ANY is None:
    _ANY = pltpu.MemorySpace.ANY
_SMEM = getattr(pltpu, "SMEM", None)
if _SMEM is None:
    _SMEM = pltpu.MemorySpace.SMEM
_CompilerParams = getattr(pltpu, "CompilerParams", None) or getattr(
    pltpu, "TPUCompilerParams"
)


def _gemm(x, w):
    m, k = x.shape
    k2, n = w.shape
    assert k == k2
    bm, bn = 512, 1024
    grid = (m // bm, n // bn)

    def body(x_ref, w_ref, o_ref):
        o_ref[...] = jnp.dot(
            x_ref[...], w_ref[...], preferred_element_type=jnp.float32
        )

    return pl.pallas_call(
        body,
        grid=grid,
        in_specs=[
            pl.BlockSpec((bm, k), lambda i, j: (i, 0)),
            pl.BlockSpec((k, bn), lambda i, j: (0, j)),
        ],
        out_specs=pl.BlockSpec((bm, bn), lambda i, j: (i, j)),
        out_shape=jax.ShapeDtypeStruct((m, n), jnp.float32),
    )(x, w)


def _all_reduce(partial):
    m, n = partial.shape
    ch = m // N_DEV
    br = 256
    nb = ch // br

    def body(p_ref, y_ref, amax_ref, v_a, v_r, v_l, s_send, s_recv, s_cp, credit, smax):
        my = lax.axis_index("i")
        left = lax.rem(my + (N_DEV - 1), N_DEV)
        right = lax.rem(my + 1, N_DEV)

        barrier = pltpu.get_barrier_semaphore()
        for nbr in (left, right):
            _sem_signal(barrier, inc=1, device_id=(nbr,), device_id_type=_DevIdType.MESH)
        _sem_wait(barrier, 2)

        smax[0, 0] = 0.0

        def rows(c, b):
            return pl.ds(c * ch + b * br, br)

        k = 0
        for b in range(nb):
            for s in range(N_DEV - 1):
                c_send = lax.rem(my + (N_DEV - s), N_DEV)
                c_recv = lax.rem(my + (N_DEV - s - 1), N_DEV)
                if s == 0:
                    cp = pltpu.make_async_copy(
                        p_ref.at[rows(c_send, b), :], v_a, s_cp
                    )
                    cp.start()
                    cp.wait()
                if k >= 2:
                    _sem_wait(credit, 1)
                rdma = pltpu.make_async_remote_copy(
                    src_ref=v_a,
                    dst_ref=v_r.at[k % 2],
                    send_sem=s_send.at[k % 2],
                    recv_sem=s_recv.at[k % 2],
                    device_id=(right,),
                    device_id_type=_DevIdType.MESH,
                )
                rdma.start()
                cp2 = pltpu.make_async_copy(p_ref.at[rows(c_recv, b), :], v_l, s_cp)
                cp2.start()
                cp2.wait()
                rdma.wait()
                v_a[...] = v_r[k % 2] + v_l[...]
                _sem_signal(credit, inc=1, device_id=(left,), device_id_type=_DevIdType.MESH)
                k += 1
            c_own = lax.rem(my + 1, N_DEV)
            st = pltpu.make_async_copy(v_a, y_ref.at[rows(c_own, b), :], s_cp)
            st.start()
            st.wait()
            smax[0, 0] = jnp.maximum(smax[0, 0], jnp.max(jnp.abs(v_a[...])))

        for h in range(N_DEV - 1):
            c_as = lax.rem(my + (N_DEV + 1 - h), N_DEV)
            c_ar = lax.rem(my + (N_DEV - h), N_DEV)
            for b in range(nb):
                cp = pltpu.make_async_copy(y_ref.at[rows(c_as, b), :], v_a, s_cp)
                cp.start()
                cp.wait()
                _sem_wait(credit, 1)
                rdma = pltpu.make_async_remote_copy(
                    src_ref=v_a,
                    dst_ref=v_r.at[k % 2],
                    send_sem=s_send.at[k % 2],
                    recv_sem=s_recv.at[k % 2],
                    device_id=(right,),
                    device_id_type=_DevIdType.MESH,
                )
                rdma.start()
                rdma.wait()
                st = pltpu.make_async_copy(v_r.at[k % 2], y_ref.at[rows(c_ar, b), :], s_cp)
                st.start()
                st.wait()
                smax[0, 0] = jnp.maximum(smax[0, 0], jnp.max(jnp.abs(v_r[k % 2])))
                _sem_signal(credit, inc=1, device_id=(left,), device_id_type=_DevIdType.MESH)
                k += 1

        amax_ref[0, 0] = smax[0, 0]
        _sem_wait(credit, 2)

    y, amax = pl.pallas_call(
        body,
        in_specs=[pl.BlockSpec(memory_space=_ANY)],
        out_specs=[
            pl.BlockSpec(memory_space=_ANY),
            pl.BlockSpec(memory_space=_SMEM),
        ],
        out_shape=[
            jax.ShapeDtypeStruct((m, n), jnp.float32),
            jax.ShapeDtypeStruct((1, 1), jnp.float32),
        ],
        scratch_shapes=[
            pltpu.VMEM((br, n), jnp.float32),
            pltpu.VMEM((2, br, n), jnp.float32),
            pltpu.VMEM((br, n), jnp.float32),
            pltpu.SemaphoreType.DMA((2,)),
            pltpu.SemaphoreType.DMA((2,)),
            pltpu.SemaphoreType.DMA,
            pltpu.SemaphoreType.REGULAR,
            pltpu.SMEM((1, 1), jnp.float32),
        ],
        compiler_params=_CompilerParams(collective_id=0),
    )(partial)
    return y, amax


def _quant_epilogue(y, amax):
    m, n = y.shape
    bm = 256
    grid = (m // bm,)

    def body(amax_ref, y_ref, o_ref):
        a = amax_ref[0, 0]
        scale = a / 448.0
        t = y_ref[...] / scale
        q = t.astype(jnp.float8_e4m3fn).astype(jnp.float32)
        o_ref[...] = q * scale

    return pl.pallas_call(
        body,
        grid=grid,
        in_specs=[
            pl.BlockSpec(memory_space=_SMEM),
            pl.BlockSpec((bm, n), lambda i: (i, 0)),
        ],
        out_specs=pl.BlockSpec((bm, n), lambda i: (i, 0)),
        out_shape=jax.ShapeDtypeStruct((m, n), jnp.float32),
    )(amax, y)


def kernel(x, w_mat):
    partial = _gemm(x, w_mat)
    y, amax = _all_reduce(partial)
    return _quant_epilogue(y, amax)


# baseline (device time: 2636584 ns/iter reference)
import jax
import jax.numpy as jnp
from jax import lax
from jax.experimental import pallas as pl
from jax.experimental.pallas import tpu as pltpu

N_DEV = 4

_sem_signal = getattr(pl, "semaphore_signal", None) or getattr(pltpu, "semaphore_signal")
_sem_wait = getattr(pl, "semaphore_wait", None) or getattr(pltpu, "semaphore_wait")
_DevIdType = getattr(pl, "DeviceIdType", None) or getattr(pltpu, "DeviceIdType")
_ANY = getattr(pltpu, "ANY", None) or pl.ANY
_SMEM = getattr(pltpu, "SMEM", None) or pltpu.MemorySpace.SMEM
_CompilerParams = getattr(pltpu, "CompilerParams", None) or getattr(
    pltpu, "TPUCompilerParams"
)


def _gemm(x, w):
    m, k = x.shape
    k2, n = w.shape
    assert k == k2
    bm, bn = 512, 1024
    grid = (m // bm, n // bn)

    def body(x_ref, w_ref, o_ref):
        o_ref[...] = jnp.dot(
            x_ref[...], w_ref[...], preferred_element_type=jnp.float32
        )

    return pl.pallas_call(
        body,
        grid=grid,
        in_specs=[
            pl.BlockSpec((bm, k), lambda i, j: (i, 0)),
            pl.BlockSpec((k, bn), lambda i, j: (0, j)),
        ],
        out_specs=pl.BlockSpec((bm, bn), lambda i, j: (i, j)),
        out_shape=jax.ShapeDtypeStruct((m, n), jnp.float32),
    )(x, w)


def _all_reduce(partial):
    m, n = partial.shape
    ch = m // N_DEV
    br = 128
    nb = ch // br

    def body(p_ref, y_ref, amax_ref, v_a, v_r, v_l, s_send, s_recv, s_cp, credit, smax):
        my = lax.axis_index("i")
        left = lax.rem(my + (N_DEV - 1), N_DEV)
        right = lax.rem(my + 1, N_DEV)

        barrier = pltpu.get_barrier_semaphore()
        for nbr in (left, right):
            _sem_signal(barrier, inc=1, device_id=(nbr,), device_id_type=_DevIdType.MESH)
        _sem_wait(barrier, 2)

        smax[0, 0] = 0.0

        def rows(c, b):
            return pl.ds(c * ch + b * br, br)

        k = 0
        for b in range(nb):
            for s in range(N_DEV - 1):
                c_send = lax.rem(my + (N_DEV - s), N_DEV)
                c_recv = lax.rem(my + (N_DEV - s - 1), N_DEV)
                if s == 0:
                    cp = pltpu.make_async_copy(
                        p_ref.at[rows(c_send, b), :], v_a, s_cp
                    )
                    cp.start()
                    cp.wait()
                if k >= 2:
                    _sem_wait(credit, 1)
                rdma = pltpu.make_async_remote_copy(
                    src_ref=v_a,
                    dst_ref=v_r.at[k % 2],
                    send_sem=s_send.at[k % 2],
                    recv_sem=s_recv.at[k % 2],
                    device_id=(right,),
                    device_id_type=_DevIdType.MESH,
                )
                rdma.start()
                cp2 = pltpu.make_async_copy(p_ref.at[rows(c_recv, b), :], v_l, s_cp)
                cp2.start()
                cp2.wait()
                rdma.wait()
                v_a[...] = v_r[k % 2] + v_l[...]
                _sem_signal(credit, inc=1, device_id=(left,), device_id_type=_DevIdType.MESH)
                k += 1
            c_own = lax.rem(my + 1, N_DEV)
            st = pltpu.make_async_copy(v_a, y_ref.at[rows(c_own, b), :], s_cp)
            st.start()
            st.wait()
            smax[0, 0] = jnp.maximum(smax[0, 0], jnp.max(jnp.abs(v_a[...])))

        for h in range(N_DEV - 1):
            c_as = lax.rem(my + (N_DEV + 1 - h), N_DEV)
            c_ar = lax.rem(my + (N_DEV - h), N_DEV)
            for b in range(nb):
                cp = pltpu.make_async_copy(y_ref.at[rows(c_as, b), :], v_a, s_cp)
                cp.start()
                cp.wait()
                _sem_wait(credit, 1)
                rdma = pltpu.make_async_remote_copy(
                    src_ref=v_a,
                    dst_ref=v_r.at[k % 2],
                    send_sem=s_send.at[k % 2],
                    recv_sem=s_recv.at[k % 2],
                    device_id=(right,),
                    device_id_type=_DevIdType.MESH,
                )
                rdma.start()
                rdma.wait()
                st = pltpu.make_async_copy(v_r.at[k % 2], y_ref.at[rows(c_ar, b), :], s_cp)
                st.start()
                st.wait()
                smax[0, 0] = jnp.maximum(smax[0, 0], jnp.max(jnp.abs(v_r[k % 2])))
                _sem_signal(credit, inc=1, device_id=(left,), device_id_type=_DevIdType.MESH)
                k += 1

        amax_ref[0, 0] = smax[0, 0]
        _sem_wait(credit, 2)

    y, amax = pl.pallas_call(
        body,
        in_specs=[pl.BlockSpec(memory_space=_ANY)],
        out_specs=[
            pl.BlockSpec(memory_space=_ANY),
            pl.BlockSpec(memory_space=_SMEM),
        ],
        out_shape=[
            jax.ShapeDtypeStruct((m, n), jnp.float32),
            jax.ShapeDtypeStruct((1, 1), jnp.float32),
        ],
        scratch_shapes=[
            pltpu.VMEM((br, n), jnp.float32),
            pltpu.VMEM((2, br, n), jnp.float32),
            pltpu.VMEM((br, n), jnp.float32),
            pltpu.SemaphoreType.DMA((2,)),
            pltpu.SemaphoreType.DMA((2,)),
            pltpu.SemaphoreType.DMA,
            pltpu.SemaphoreType.REGULAR,
            pltpu.SMEM((1, 1), jnp.float32),
        ],
        compiler_params=_CompilerParams(collective_id=0),
    )(partial)
    return y, amax


def _quant_epilogue(y, amax):
    m, n = y.shape
    bm = 128
    grid = (m // bm,)

    def body(amax_ref, y_ref, o_ref):
        a = amax_ref[0, 0]
        scale = a / 448.0
        t = y_ref[...] / scale
        q = t.astype(jnp.float8_e4m3fn).astype(jnp.float32)
        o_ref[...] = q * scale

    return pl.pallas_call(
        body,
        grid=grid,
        in_specs=[
            pl.BlockSpec(memory_space=_SMEM),
            pl.BlockSpec((bm, n), lambda i: (i, 0)),
        ],
        out_specs=pl.BlockSpec((bm, n), lambda i: (i, 0)),
        out_shape=jax.ShapeDtypeStruct((m, n), jnp.float32),
    )(amax, y)


def kernel(x, w_mat):
    partial = _gemm(x, w_mat)
    y, amax = _all_reduce(partial)
    return _quant_epilogue(y, amax)


# device time: 1237203 ns/iter; 2.1311x vs baseline; 2.1311x over previous
import jax
import jax.numpy as jnp
from jax import lax
from jax.experimental import pallas as pl
from jax.experimental.pallas import tpu as pltpu

N_DEV = 4

_sem_signal = getattr(pl, "semaphore_signal", None) or getattr(pltpu, "semaphore_signal")
_sem_wait = getattr(pl, "semaphore_wait", None) or getattr(pltpu, "semaphore_wait")
_DevIdType = getattr(pl, "DeviceIdType", None) or getattr(pltpu, "DeviceIdType")
_ANY = getattr(pltpu, "ANY", None) or pl.ANY
_SMEM = getattr(pltpu, "SMEM", None) or pltpu.MemorySpace.SMEM
_CompilerParams = getattr(pltpu, "CompilerParams", None) or getattr(
    pltpu, "TPUCompilerParams"
)


def _gemm(x, w):
    m, k = x.shape
    k2, n = w.shape
    assert k == k2
    bm, bn = 512, 1024
    grid = (m // bm, n // bn)

    def body(x_ref, w_ref, o_ref):
        o_ref[...] = jnp.dot(
            x_ref[...], w_ref[...], preferred_element_type=jnp.float32
        )

    return pl.pallas_call(
        body,
        grid=grid,
        in_specs=[
            pl.BlockSpec((bm, k), lambda i, j: (i, 0)),
            pl.BlockSpec((k, bn), lambda i, j: (0, j)),
        ],
        out_specs=pl.BlockSpec((bm, bn), lambda i, j: (i, j)),
        out_shape=jax.ShapeDtypeStruct((m, n), jnp.float32),
    )(x, w)


def _all_reduce_quant(partial):
    m, n = partial.shape
    ch = m // N_DEV
    br = 128
    nb = ch // br

    def body(p_ref, q_ref, amax_ref, yown_ref,
             v_a, v_ab, v_rb, v_l, v_ms, v_mr, v_qs, v_qr,
             s_send, s_recv, s_cp, credit, smax):
        my = lax.axis_index("i")
        left = lax.rem(my + (N_DEV - 1), N_DEV)
        right = lax.rem(my + 1, N_DEV)
        c_own = lax.rem(my + 1, N_DEV)

        barrier = pltpu.get_barrier_semaphore()
        for nbr in (left, right):
            _sem_signal(barrier, inc=1, device_id=(nbr,), device_id_type=_DevIdType.MESH)
        _sem_wait(barrier, 2)

        smax[0, 0] = 0.0

        def rows(c, b):
            return pl.ds(c * ch + b * br, br)

        def ring_send(src, dst, slot):
            return pltpu.make_async_remote_copy(
                src_ref=src,
                dst_ref=dst.at[slot],
                send_sem=s_send.at[slot],
                recv_sem=s_recv.at[slot],
                device_id=(right,),
                device_id_type=_DevIdType.MESH,
            )

        def give_credit():
            _sem_signal(credit, inc=1, device_id=(left,), device_id_type=_DevIdType.MESH)

        k = 0
        for b in range(nb):
            for s in range(N_DEV - 1):
                c_send = lax.rem(my + (N_DEV - s), N_DEV)
                c_recv = lax.rem(my + (N_DEV - s - 1), N_DEV)
                if s == 0:
                    cp = pltpu.make_async_copy(p_ref.at[rows(c_send, b), :], v_a, s_cp)
                    cp.start()
                    cp.wait()
                    v_ab[...] = v_a[...].astype(jnp.bfloat16)
                if k >= 2:
                    _sem_wait(credit, 1)
                rdma = ring_send(v_ab, v_rb, k % 2)
                rdma.start()
                cp2 = pltpu.make_async_copy(p_ref.at[rows(c_recv, b), :], v_l, s_cp)
                cp2.start()
                cp2.wait()
                rdma.wait()
                v_a[...] = v_rb[k % 2].astype(jnp.float32) + v_l[...]
                if s < N_DEV - 2:
                    v_ab[...] = v_a[...].astype(jnp.bfloat16)
                give_credit()
                k += 1
            st = pltpu.make_async_copy(v_a, yown_ref.at[pl.ds(b * br, br), :], s_cp)
            st.start()
            st.wait()
            smax[0, 0] = jnp.maximum(smax[0, 0], jnp.max(jnp.abs(v_a[...])))

        v_ms[...] = jnp.full((8, 128), smax[0, 0], jnp.float32)
        for h in range(N_DEV - 1):
            _sem_wait(credit, 1)
            rdma = ring_send(v_ms, v_mr, k % 2)
            rdma.start()
            rdma.wait()
            v_ms[...] = jnp.maximum(v_ms[...], v_mr[k % 2])
            give_credit()
            k += 1
        amax_ref[0, 0] = jnp.max(v_ms[...])

        inv = 448.0 / amax_ref[0, 0]
        for b in range(nb):
            cp = pltpu.make_async_copy(yown_ref.at[pl.ds(b * br, br), :], v_l, s_cp)
            cp.start()
            cp.wait()
            v_qs[...] = (v_l[...] * inv).astype(jnp.float8_e4m3fn)
            st = pltpu.make_async_copy(v_qs, q_ref.at[rows(c_own, b), :], s_cp)
            st.start()
            st.wait()

        for h in range(N_DEV - 1):
            c_as = lax.rem(my + (N_DEV + 1 - h), N_DEV)
            c_ar = lax.rem(my + (N_DEV - h), N_DEV)
            for b in range(nb):
                cp = pltpu.make_async_copy(q_ref.at[rows(c_as, b), :], v_qs, s_cp)
                cp.start()
                cp.wait()
                _sem_wait(credit, 1)
                rdma = ring_send(v_qs, v_qr, k % 2)
                rdma.start()
                rdma.wait()
                st = pltpu.make_async_copy(v_qr.at[k % 2], q_ref.at[rows(c_ar, b), :], s_cp)
                st.start()
                st.wait()
                give_credit()
                k += 1

        _sem_wait(credit, 2)

    q, amax, _yown = pl.pallas_call(
        body,
        in_specs=[pl.BlockSpec(memory_space=_ANY)],
        out_specs=[
            pl.BlockSpec(memory_space=_ANY),
            pl.BlockSpec(memory_space=_SMEM),
            pl.BlockSpec(memory_space=_ANY),
        ],
        out_shape=[
            jax.ShapeDtypeStruct((m, n), jnp.float8_e4m3fn),
            jax.ShapeDtypeStruct((1, 1), jnp.float32),
            jax.ShapeDtypeStruct((ch, n), jnp.float32),
        ],
        scratch_shapes=[
            pltpu.VMEM((br, n), jnp.float32),
            pltpu.VMEM((br, n), jnp.bfloat16),
            pltpu.VMEM((2, br, n), jnp.bfloat16),
            pltpu.VMEM((br, n), jnp.float32),
            pltpu.VMEM((8, 128), jnp.float32),
            pltpu.VMEM((2, 8, 128), jnp.float32),
            pltpu.VMEM((br, n), jnp.float8_e4m3fn),
            pltpu.VMEM((2, br, n), jnp.float8_e4m3fn),
            pltpu.SemaphoreType.DMA((2,)),
            pltpu.SemaphoreType.DMA((2,)),
            pltpu.SemaphoreType.DMA,
            pltpu.SemaphoreType.REGULAR,
            pltpu.SMEM((1, 1), jnp.float32),
        ],
        compiler_params=_CompilerParams(collective_id=0),
    )(partial)
    return q, amax


def _dequant(q, amax):
    m, n = q.shape
    bm = 256
    grid = (m // bm,)

    def body(amax_ref, q_ref, o_ref):
        scale = amax_ref[0, 0] / 448.0
        o_ref[...] = q_ref[...].astype(jnp.float32) * scale

    return pl.pallas_call(
        body,
        grid=grid,
        in_specs=[
            pl.BlockSpec(memory_space=_SMEM),
            pl.BlockSpec((bm, n), lambda i: (i, 0)),
        ],
        out_specs=pl.BlockSpec((bm, n), lambda i: (i, 0)),
        out_shape=jax.ShapeDtypeStruct((m, n), jnp.float32),
    )(amax, q)


def kernel(x, w_mat):
    partial = _gemm(x, w_mat)
    q, amax = _all_reduce_quant(partial)
    return _dequant(q, amax)


# device time: 821994 ns/iter; 3.2075x vs baseline; 1.5051x over previous
import jax
import jax.numpy as jnp
from jax import lax
from jax.experimental import pallas as pl
from jax.experimental.pallas import tpu as pltpu

N_DEV = 4

_sem_signal = getattr(pl, "semaphore_signal", None) or getattr(pltpu, "semaphore_signal")
_sem_wait = getattr(pl, "semaphore_wait", None) or getattr(pltpu, "semaphore_wait")
_DevIdType = getattr(pl, "DeviceIdType", None) or getattr(pltpu, "DeviceIdType")
_ANY = getattr(pltpu, "ANY", None) or pl.ANY
_SMEM = getattr(pltpu, "SMEM", None) or pltpu.MemorySpace.SMEM
_CompilerParams = getattr(pltpu, "CompilerParams", None) or getattr(
    pltpu, "TPUCompilerParams"
)


def _gemm(x, w):
    m, k = x.shape
    k2, n = w.shape
    assert k == k2
    bm, bn = 512, 1024
    grid = (m // bm, n // bn)

    def body(x_ref, w_ref, o_ref):
        o_ref[...] = jnp.dot(
            x_ref[...], w_ref[...], preferred_element_type=jnp.float32
        )

    return pl.pallas_call(
        body,
        grid=grid,
        in_specs=[
            pl.BlockSpec((bm, k), lambda i, j: (i, 0)),
            pl.BlockSpec((k, bn), lambda i, j: (0, j)),
        ],
        out_specs=pl.BlockSpec((bm, bn), lambda i, j: (i, j)),
        out_shape=jax.ShapeDtypeStruct((m, n), jnp.float32),
    )(x, w)


def _all_reduce_quant(partial):
    m, n = partial.shape
    ch = m // N_DEV
    hn = n // 2
    br = 256
    nb = ch // br

    def body(p_ref, o_ref, yown_ref,
             v_aR, v_aL, v_abR, v_abL, v_rbR, v_rbL,
             v_ms, v_mr, v_qsR, v_qsL, v_qrR, v_qrL,
             s_sR, s_rR, s_sL, s_rL, s_cp, s_cp2, credR, credL, smax):
        my = lax.axis_index("i")
        left = lax.rem(my + (N_DEV - 1), N_DEV)
        right = lax.rem(my + 1, N_DEV)
        c_ownR = lax.rem(my + 1, N_DEV)
        c_ownL = lax.rem(my + (N_DEV - 1), N_DEV)

        barrier = pltpu.get_barrier_semaphore()
        for nbr in (left, right):
            _sem_signal(barrier, inc=1, device_id=(nbr,), device_id_type=_DevIdType.MESH)
        _sem_wait(barrier, 2)

        smax[0, 0] = 0.0

        def rowsR(c, b):
            return (pl.ds(c * ch + b * br, br), pl.ds(0, hn))

        def rowsL(c, b):
            return (pl.ds(c * ch + b * br, br), pl.ds(hn, hn))

        def send_R(src, dst, slot):
            return pltpu.make_async_remote_copy(
                src_ref=src, dst_ref=dst.at[slot],
                send_sem=s_sR.at[slot], recv_sem=s_rR.at[slot],
                device_id=(right,), device_id_type=_DevIdType.MESH,
            )

        def send_L(src, dst, slot):
            return pltpu.make_async_remote_copy(
                src_ref=src, dst_ref=dst.at[slot],
                send_sem=s_sL.at[slot], recv_sem=s_rL.at[slot],
                device_id=(left,), device_id_type=_DevIdType.MESH,
            )

        def credits(k):
            if k >= 2:
                _sem_wait(credR, 1)
                _sem_wait(credL, 1)

        def give_credits():
            _sem_signal(credR, inc=1, device_id=(left,), device_id_type=_DevIdType.MESH)
            _sem_signal(credL, inc=1, device_id=(right,), device_id_type=_DevIdType.MESH)

        k = 0
        for b in range(nb):
            for s in range(N_DEV - 1):
                cR_send = lax.rem(my + (N_DEV - s), N_DEV)
                cR_recv = lax.rem(my + (N_DEV - s - 1), N_DEV)
                cL_send = lax.rem(my + s, N_DEV)
                cL_recv = lax.rem(my + s + 1, N_DEV)
                if s == 0:
                    cp = pltpu.make_async_copy(p_ref.at[rowsR(cR_send, b)], v_aR, s_cp)
                    cp2 = pltpu.make_async_copy(p_ref.at[rowsL(cL_send, b)], v_aL, s_cp2)
                    cp.start()
                    cp2.start()
                    cp.wait()
                    cp2.wait()
                    v_abR[...] = v_aR[...].astype(jnp.bfloat16)
                    v_abL[...] = v_aL[...].astype(jnp.bfloat16)
                credits(k)
                rR = send_R(v_abR, v_rbR, k % 2)
                rL = send_L(v_abL, v_rbL, k % 2)
                rR.start()
                rL.start()
                cp = pltpu.make_async_copy(p_ref.at[rowsR(cR_recv, b)], v_aR, s_cp)
                cp2 = pltpu.make_async_copy(p_ref.at[rowsL(cL_recv, b)], v_aL, s_cp2)
                cp.start()
                cp2.start()
                cp.wait()
                cp2.wait()
                rR.wait()
                rL.wait()
                v_aR[...] = v_aR[...] + v_rbR[k % 2].astype(jnp.float32)
                v_aL[...] = v_aL[...] + v_rbL[k % 2].astype(jnp.float32)
                if s < N_DEV - 2:
                    v_abR[...] = v_aR[...].astype(jnp.bfloat16)
                    v_abL[...] = v_aL[...].astype(jnp.bfloat16)
                give_credits()
                k += 1
            st = pltpu.make_async_copy(v_aR, yown_ref.at[pl.ds(b * br, br), pl.ds(0, hn)], s_cp)
            st2 = pltpu.make_async_copy(v_aL, yown_ref.at[pl.ds(b * br, br), pl.ds(hn, hn)], s_cp2)
            st.start()
            st2.start()
            st.wait()
            st2.wait()
            smax[0, 0] = jnp.maximum(
                smax[0, 0],
                jnp.maximum(jnp.max(jnp.abs(v_aR[...])), jnp.max(jnp.abs(v_aL[...]))),
            )

        v_ms[...] = jnp.full((8, 128), smax[0, 0], jnp.float32)
        for h in range(N_DEV - 1):
            if k >= 2:
                _sem_wait(credR, 1)
            rdma = pltpu.make_async_remote_copy(
                src_ref=v_ms, dst_ref=v_mr.at[k % 2],
                send_sem=s_sR.at[k % 2], recv_sem=s_rR.at[k % 2],
                device_id=(right,), device_id_type=_DevIdType.MESH,
            )
            rdma.start()
            rdma.wait()
            v_ms[...] = jnp.maximum(v_ms[...], v_mr[k % 2])
            _sem_signal(credR, inc=1, device_id=(left,), device_id_type=_DevIdType.MESH)
            k += 1
        smax[0, 0] = jnp.max(v_ms[...])

        _sem_wait(credR, 2)
        _sem_wait(credL, 2)

        inv = 448.0 / smax[0, 0]
        scale = smax[0, 0] / 448.0

        def dequant_store(qref, vdst, rows_idx, sem):
            vdst[...] = qref[...].astype(jnp.float32) * scale
            st = pltpu.make_async_copy(vdst, o_ref.at[rows_idx], sem)
            st.start()
            st.wait()

        for b in range(nb):
            cp = pltpu.make_async_copy(yown_ref.at[pl.ds(b * br, br), pl.ds(0, hn)], v_aR, s_cp)
            cp2 = pltpu.make_async_copy(yown_ref.at[pl.ds(b * br, br), pl.ds(hn, hn)], v_aL, s_cp2)
            cp.start()
            cp2.start()
            cp.wait()
            cp2.wait()
            v_qsR[...] = (v_aR[...] * inv).astype(jnp.float8_e4m3fn)
            v_qsL[...] = (v_aL[...] * inv).astype(jnp.float8_e4m3fn)
            dequant_store(v_qsR, v_aR, rowsR(c_ownR, b), s_cp)
            dequant_store(v_qsL, v_aL, rowsL(c_ownL, b), s_cp2)
            for h in range(N_DEV - 1):
                j = 3 * b + h
                cR_ar = lax.rem(my + (N_DEV - h), N_DEV)
                cL_ar = lax.rem(my + h, N_DEV)
                srcR = v_qsR if h == 0 else v_qrR.at[(j + 1) % 2]
                srcL = v_qsL if h == 0 else v_qrL.at[(j + 1) % 2]
                if j >= 2:
                    _sem_wait(credR, 1)
                    _sem_wait(credL, 1)
                rR = send_R(srcR, v_qrR, j % 2)
                rL = send_L(srcL, v_qrL, j % 2)
                rR.start()
                rL.start()
                rR.wait()
                rL.wait()
                if h > 0:
                    give_credits()
                dequant_store(v_qrR.at[j % 2], v_aR, rowsR(cR_ar, b), s_cp)
                dequant_store(v_qrL.at[j % 2], v_aL, rowsL(cL_ar, b), s_cp2)
                if h == N_DEV - 2:
                    give_credits()

        _sem_wait(credR, 2)
        _sem_wait(credL, 2)

    out, _yown = pl.pallas_call(
        body,
        in_specs=[pl.BlockSpec(memory_space=_ANY)],
        out_specs=[
            pl.BlockSpec(memory_space=_ANY),
            pl.BlockSpec(memory_space=_ANY),
        ],
        out_shape=[
            jax.ShapeDtypeStruct((m, n), jnp.float32),
            jax.ShapeDtypeStruct((ch, n), jnp.float32),
        ],
        scratch_shapes=[
            pltpu.VMEM((br, hn), jnp.float32),
            pltpu.VMEM((br, hn), jnp.float32),
            pltpu.VMEM((br, hn), jnp.bfloat16),
            pltpu.VMEM((br, hn), jnp.bfloat16),
            pltpu.VMEM((2, br, hn), jnp.bfloat16),
            pltpu.VMEM((2, br, hn), jnp.bfloat16),
            pltpu.VMEM((8, 128), jnp.float32),
            pltpu.VMEM((2, 8, 128), jnp.float32),
            pltpu.VMEM((br, hn), jnp.float8_e4m3fn),
            pltpu.VMEM((br, hn), jnp.float8_e4m3fn),
            pltpu.VMEM((2, br, hn), jnp.float8_e4m3fn),
            pltpu.VMEM((2, br, hn), jnp.float8_e4m3fn),
            pltpu.SemaphoreType.DMA((2,)),
            pltpu.SemaphoreType.DMA((2,)),
            pltpu.SemaphoreType.DMA((2,)),
            pltpu.SemaphoreType.DMA((2,)),
            pltpu.SemaphoreType.DMA,
            pltpu.SemaphoreType.DMA,
            pltpu.SemaphoreType.REGULAR,
            pltpu.SemaphoreType.REGULAR,
            pltpu.SMEM((1, 1), jnp.float32),
        ],
        compiler_params=_CompilerParams(collective_id=0),
    )(partial)
    return out


def kernel(x, w_mat):
    partial = _gemm(x, w_mat)
    return _all_reduce_quant(partial)


# device time: 724889 ns/iter; 3.6372x vs baseline; 1.1340x over previous
import jax
import jax.numpy as jnp
from jax import lax
from jax.experimental import pallas as pl
from jax.experimental.pallas import tpu as pltpu

N_DEV = 4

_sem_signal = getattr(pl, "semaphore_signal", None) or getattr(pltpu, "semaphore_signal")
_sem_wait = getattr(pl, "semaphore_wait", None) or getattr(pltpu, "semaphore_wait")
_DevIdType = getattr(pl, "DeviceIdType", None) or getattr(pltpu, "DeviceIdType")
_ANY = getattr(pltpu, "ANY", None) or pl.ANY
_SMEM = getattr(pltpu, "SMEM", None) or pltpu.MemorySpace.SMEM
_CompilerParams = getattr(pltpu, "CompilerParams", None) or getattr(
    pltpu, "TPUCompilerParams"
)

_WIRE = jnp.bfloat16


def _gemm_rs(x, w):
    m, k = x.shape
    k2, n = w.shape
    assert k == k2
    ch = m // N_DEV
    hn = n // 2
    br = 256
    nb = ch // br
    tw = 1024
    ntw = hn // tw

    def body(x_ref, w_ref, yown_ref, smax_ref,
             v_xR, v_xL, v_w, v_aR, v_aL, v_abR, v_abL, v_rbR, v_rbL,
             s_sR, s_rR, s_sL, s_rL, s_x, s_x2, s_w, s_st, s_st2,
             credR, credL, smax):
        my = lax.axis_index("i")
        left = lax.rem(my + (N_DEV - 1), N_DEV)
        right = lax.rem(my + 1, N_DEV)

        barrier = pltpu.get_barrier_semaphore()
        for nbr in (left, right):
            _sem_signal(barrier, inc=1, device_id=(nbr,), device_id_type=_DevIdType.MESH)
        _sem_wait(barrier, 2)

        smax[0, 0] = 0.0

        def gemm_half(c, half, v_x, v_a, sx, b):
            cpx = pltpu.make_async_copy(
                x_ref.at[pl.ds(c * ch + b * br, br), :], v_x, sx
            )
            cpx.start()
            ld = pltpu.make_async_copy(
                w_ref.at[:, pl.ds(half * hn, tw)], v_w.at[0], s_w.at[0]
            )
            ld.start()
            cpx.wait()
            for nt in range(ntw):
                if nt + 1 < ntw:
                    nxt = pltpu.make_async_copy(
                        w_ref.at[:, pl.ds(half * hn + (nt + 1) * tw, tw)],
                        v_w.at[(nt + 1) % 2],
                        s_w.at[(nt + 1) % 2],
                    )
                    nxt.start()
                pltpu.make_async_copy(
                    w_ref.at[:, pl.ds(half * hn + nt * tw, tw)],
                    v_w.at[nt % 2],
                    s_w.at[nt % 2],
                ).wait()
                v_a[:, nt * tw:(nt + 1) * tw] = jnp.dot(
                    v_x[...], v_w[nt % 2], preferred_element_type=jnp.float32
                )

        def send_R(src, dst, slot):
            return pltpu.make_async_remote_copy(
                src_ref=src, dst_ref=dst.at[slot],
                send_sem=s_sR.at[slot], recv_sem=s_rR.at[slot],
                device_id=(right,), device_id_type=_DevIdType.MESH,
            )

        def send_L(src, dst, slot):
            return pltpu.make_async_remote_copy(
                src_ref=src, dst_ref=dst.at[slot],
                send_sem=s_sL.at[slot], recv_sem=s_rL.at[slot],
                device_id=(left,), device_id_type=_DevIdType.MESH,
            )

        k = 0
        for b in range(nb):
            for s in range(N_DEV - 1):
                cR_send = lax.rem(my + (N_DEV - s), N_DEV)
                cR_recv = lax.rem(my + (N_DEV - s - 1), N_DEV)
                cL_send = lax.rem(my + s, N_DEV)
                cL_recv = lax.rem(my + s + 1, N_DEV)
                if s == 0:
                    gemm_half(cR_send, 0, v_xR, v_aR, s_x, b)
                    gemm_half(cL_send, 1, v_xL, v_aL, s_x2, b)
                    v_abR[...] = v_aR[...].astype(_WIRE)
                    v_abL[...] = v_aL[...].astype(_WIRE)
                if k >= 2:
                    _sem_wait(credR, 1)
                    _sem_wait(credL, 1)
                rR = send_R(v_abR, v_rbR, k % 2)
                rL = send_L(v_abL, v_rbL, k % 2)
                rR.start()
                rL.start()
                gemm_half(cR_recv, 0, v_xR, v_aR, s_x, b)
                gemm_half(cL_recv, 1, v_xL, v_aL, s_x2, b)
                rR.wait()
                rL.wait()
                v_aR[...] = v_aR[...] + v_rbR[k % 2].astype(jnp.float32)
                v_aL[...] = v_aL[...] + v_rbL[k % 2].astype(jnp.float32)
                if s < N_DEV - 2:
                    v_abR[...] = v_aR[...].astype(_WIRE)
                    v_abL[...] = v_aL[...].astype(_WIRE)
                _sem_signal(credR, inc=1, device_id=(left,), device_id_type=_DevIdType.MESH)
                _sem_signal(credL, inc=1, device_id=(right,), device_id_type=_DevIdType.MESH)
                k += 1
            st = pltpu.make_async_copy(v_aR, yown_ref.at[pl.ds(b * br, br), pl.ds(0, hn)], s_st)
            st2 = pltpu.make_async_copy(v_aL, yown_ref.at[pl.ds(b * br, br), pl.ds(hn, hn)], s_st2)
            st.start()
            st2.start()
            st.wait()
            st2.wait()
            smax[0, 0] = jnp.maximum(
                smax[0, 0],
                jnp.maximum(jnp.max(jnp.abs(v_aR[...])), jnp.max(jnp.abs(v_aL[...]))),
            )

        smax_ref[0, 0] = smax[0, 0]
        _sem_wait(credR, 2)
        _sem_wait(credL, 2)

    return pl.pallas_call(
        body,
        in_specs=[
            pl.BlockSpec(memory_space=_ANY),
            pl.BlockSpec(memory_space=_ANY),
        ],
        out_specs=[
            pl.BlockSpec(memory_space=_ANY),
            pl.BlockSpec(memory_space=_SMEM),
        ],
        out_shape=[
            jax.ShapeDtypeStruct((ch, n), jnp.float32),
            jax.ShapeDtypeStruct((1, 1), jnp.float32),
        ],
        scratch_shapes=[
            pltpu.VMEM((br, k), jnp.float32),
            pltpu.VMEM((br, k), jnp.float32),
            pltpu.VMEM((2, k, tw), jnp.float32),
            pltpu.VMEM((br, hn), jnp.float32),
            pltpu.VMEM((br, hn), jnp.float32),
            pltpu.VMEM((br, hn), _WIRE),
            pltpu.VMEM((br, hn), _WIRE),
            pltpu.VMEM((2, br, hn), _WIRE),
            pltpu.VMEM((2, br, hn), _WIRE),
            pltpu.SemaphoreType.DMA((2,)),
            pltpu.SemaphoreType.DMA((2,)),
            pltpu.SemaphoreType.DMA((2,)),
            pltpu.SemaphoreType.DMA((2,)),
            pltpu.SemaphoreType.DMA,
            pltpu.SemaphoreType.DMA,
            pltpu.SemaphoreType.DMA((2,)),
            pltpu.SemaphoreType.DMA,
            pltpu.SemaphoreType.DMA,
            pltpu.SemaphoreType.REGULAR,
            pltpu.SemaphoreType.REGULAR,
            pltpu.SMEM((1, 1), jnp.float32),
        ],
        compiler_params=_CompilerParams(collective_id=0),
    )(x, w)


def _amax_quant_ag(yown, smax_local):
    ch, n = yown.shape
    m = ch * N_DEV
    hn = n // 2
    br = 256
    nb = ch // br

    def body(yown_ref, smax_ref, o_ref,
             v_aR, v_aL, v_ms, v_mr, v_qsR, v_qsL, v_qrR, v_qrL,
             s_sR, s_rR, s_sL, s_rL, s_cp, s_cp2, credR, credL, smax):
        my = lax.axis_index("i")
        left = lax.rem(my + (N_DEV - 1), N_DEV)
        right = lax.rem(my + 1, N_DEV)
        c_ownR = lax.rem(my + 1, N_DEV)
        c_ownL = lax.rem(my + (N_DEV - 1), N_DEV)

        barrier = pltpu.get_barrier_semaphore()
        for nbr in (left, right):
            _sem_signal(barrier, inc=1, device_id=(nbr,), device_id_type=_DevIdType.MESH)
        _sem_wait(barrier, 2)

        def rowsR(c, b):
            return (pl.ds(c * ch + b * br, br), pl.ds(0, hn))

        def rowsL(c, b):
            return (pl.ds(c * ch + b * br, br), pl.ds(hn, hn))

        def send_R(src, dst, slot):
            return pltpu.make_async_remote_copy(
                src_ref=src, dst_ref=dst.at[slot],
                send_sem=s_sR.at[slot], recv_sem=s_rR.at[slot],
                device_id=(right,), device_id_type=_DevIdType.MESH,
            )

        def send_L(src, dst, slot):
            return pltpu.make_async_remote_copy(
                src_ref=src, dst_ref=dst.at[slot],
                send_sem=s_sL.at[slot], recv_sem=s_rL.at[slot],
                device_id=(left,), device_id_type=_DevIdType.MESH,
            )

        def give_credits():
            _sem_signal(credR, inc=1, device_id=(left,), device_id_type=_DevIdType.MESH)
            _sem_signal(credL, inc=1, device_id=(right,), device_id_type=_DevIdType.MESH)

        v_ms[...] = jnp.full((8, 128), smax_ref[0, 0], jnp.float32)
        for h in range(N_DEV - 1):
            if h >= 2:
                _sem_wait(credR, 1)
            rdma = pltpu.make_async_remote_copy(
                src_ref=v_ms, dst_ref=v_mr.at[h % 2],
                send_sem=s_sR.at[h % 2], recv_sem=s_rR.at[h % 2],
                device_id=(right,), device_id_type=_DevIdType.MESH,
            )
            rdma.start()
            rdma.wait()
            v_ms[...] = jnp.maximum(v_ms[...], v_mr[h % 2])
            _sem_signal(credR, inc=1, device_id=(left,), device_id_type=_DevIdType.MESH)
        smax[0, 0] = jnp.max(v_ms[...])
        _sem_wait(credR, 2)

        inv = 448.0 / smax[0, 0]
        scale = smax[0, 0] / 448.0

        def dequant_store(qref, vdst, rows_idx, sem):
            vdst[...] = qref[...].astype(jnp.float32) * scale
            st = pltpu.make_async_copy(vdst, o_ref.at[rows_idx], sem)
            st.start()
            st.wait()

        for b in range(nb):
            cp = pltpu.make_async_copy(yown_ref.at[pl.ds(b * br, br), pl.ds(0, hn)], v_aR, s_cp)
            cp2 = pltpu.make_async_copy(yown_ref.at[pl.ds(b * br, br), pl.ds(hn, hn)], v_aL, s_cp2)
            cp.start()
            cp2.start()
            cp.wait()
            cp2.wait()
            v_qsR[...] = (v_aR[...] * inv).astype(jnp.float8_e4m3fn)
            v_qsL[...] = (v_aL[...] * inv).astype(jnp.float8_e4m3fn)
            dequant_store(v_qsR, v_aR, rowsR(c_ownR, b), s_cp)
            dequant_store(v_qsL, v_aL, rowsL(c_ownL, b), s_cp2)
            for h in range(N_DEV - 1):
                j = 3 * b + h
                cR_ar = lax.rem(my + (N_DEV - h), N_DEV)
                cL_ar = lax.rem(my + h, N_DEV)
                srcR = v_qsR if h == 0 else v_qrR.at[(j + 1) % 2]
                srcL = v_qsL if h == 0 else v_qrL.at[(j + 1) % 2]
                if j >= 2:
                    _sem_wait(credR, 1)
                    _sem_wait(credL, 1)
                rR = send_R(srcR, v_qrR, j % 2)
                rL = send_L(srcL, v_qrL, j % 2)
                rR.start()
                rL.start()
                rR.wait()
                rL.wait()
                if h > 0:
                    give_credits()
                dequant_store(v_qrR.at[j % 2], v_aR, rowsR(cR_ar, b), s_cp)
                dequant_store(v_qrL.at[j % 2], v_aL, rowsL(cL_ar, b), s_cp2)
                if h == N_DEV - 2:
                    give_credits()

        _sem_wait(credR, 2)
        _sem_wait(credL, 2)

    return pl.pallas_call(
        body,
        in_specs=[
            pl.BlockSpec(memory_space=_ANY),
            pl.BlockSpec(memory_space=_SMEM),
        ],
        out_specs=pl.BlockSpec(memory_space=_ANY),
        out_shape=jax.ShapeDtypeStruct((m, n), jnp.float32),
        scratch_shapes=[
            pltpu.VMEM((br, hn), jnp.float32),
            pltpu.VMEM((br, hn), jnp.float32),
            pltpu.VMEM((8, 128), jnp.float32),
            pltpu.VMEM((2, 8, 128), jnp.float32),
            pltpu.VMEM((br, hn), jnp.float8_e4m3fn),
            pltpu.VMEM((br, hn), jnp.float8_e4m3fn),
            pltpu.VMEM((2, br, hn), jnp.float8_e4m3fn),
            pltpu.VMEM((2, br, hn), jnp.float8_e4m3fn),
            pltpu.SemaphoreType.DMA((2,)),
            pltpu.SemaphoreType.DMA((2,)),
            pltpu.SemaphoreType.DMA((2,)),
            pltpu.SemaphoreType.DMA((2,)),
            pltpu.SemaphoreType.DMA,
            pltpu.SemaphoreType.DMA,
            pltpu.SemaphoreType.REGULAR,
            pltpu.SemaphoreType.REGULAR,
            pltpu.SMEM((1, 1), jnp.float32),
        ],
        compiler_params=_CompilerParams(collective_id=1),
    )(yown, smax_local)


def kernel(x, w_mat):
    yown, smax_local = _gemm_rs(x, w_mat)
    return _amax_quant_ag(yown, smax_local)
